# baseline (speedup 1.0000x reference)
import functools

import jax
import jax.numpy as jnp
from jax import lax
from jax.experimental import pallas as pl
from jax.experimental.pallas import tpu as pltpu
from jax.experimental.pallas import tpu_sc as plsc

_NUM_CORES = 2
_NUM_SUBCORES = 16
_LANES = 16


def _build_a_call(q_len, v_len, max_len, d):
    p = (max_len - 1) // 2
    off = (q_len - 1) - p
    a_rows = q_len + v_len
    nw = _NUM_CORES * _NUM_SUBCORES
    assert a_rows % nw == 0
    bpw = a_rows // nw

    mesh = plsc.VectorSubcoreMesh(core_axis_name="c", subcore_axis_name="s")

    @functools.partial(
        pl.kernel,
        out_type=jax.ShapeDtypeStruct((a_rows * d,), jnp.float32),
        mesh=mesh,
        compiler_params=pltpu.CompilerParams(use_tc_tiling_on_sc=False),
        scratch_types=[
            pltpu.VMEM((max_len * d,), jnp.float32),
            pltpu.VMEM((bpw * d,), jnp.float32),
            pltpu.SemaphoreType.DMA,
        ],
    )
    def build_a(emb_hbm, a_hbm, emb_v, build_v, sem):
        wid = lax.axis_index("c") * _NUM_SUBCORES + lax.axis_index("s")
        pltpu.async_copy(emb_hbm, emb_v, sem).wait()
        bias = wid * bpw - off

        @pl.loop(0, bpw, step=4)
        def _(t):
            for u in range(4):
                k = jnp.minimum(jnp.maximum(bias + (t + u), 0), max_len - 1)
                for h in range(d // _LANES):
                    build_v[pl.ds((t + u) * d + h * _LANES, _LANES)] = (
                        emb_v[pl.ds(k * d + h * _LANES, _LANES)]
                    )

        pltpu.async_copy(build_v, a_hbm.at[pl.ds(wid * bpw * d, bpw * d)], sem).wait()

    return build_a


def _emit_rows_call(q_len, v_len, max_len, d):
    a_rows = q_len + v_len
    row_w = v_len * d
    rpc = q_len // _NUM_CORES
    nfire = 8

    mesh = plsc.ScalarSubcoreMesh(axis_name="c", num_cores=_NUM_CORES)

    @functools.partial(
        pl.kernel,
        out_type=jax.ShapeDtypeStruct((q_len, row_w), jnp.float32),
        mesh=mesh,
        compiler_params=pltpu.CompilerParams(use_tc_tiling_on_sc=False),
        scratch_types=[
            pltpu.VMEM_SHARED((a_rows * d,), jnp.float32),
            pltpu.SemaphoreType.DMA,
            pltpu.SemaphoreType.DMA,
        ],
    )
    def emit_rows(a_hbm, out_hbm, a_sh, gsem, csem):
        cid = lax.axis_index("c")
        i0 = cid * rpc
        pltpu.async_copy(a_hbm, a_sh, gsem).wait()

        def start_of(r):
            return ((q_len - 1) - (i0 + r)) * d

        for b in range(nfire):
            pltpu.async_copy(
                a_sh.at[pl.ds(start_of(b), row_w)], out_hbm.at[i0 + b], csem
            )

        @pl.loop(0, rpc - nfire)
        def _(r):
            pltpu.make_async_copy(
                a_sh.at[pl.ds(start_of(r), row_w)], out_hbm.at[i0 + r], csem
            ).wait()
            pltpu.async_copy(
                a_sh.at[pl.ds(start_of(r + nfire), row_w)],
                out_hbm.at[i0 + r + nfire],
                csem,
            )

        @pl.loop(rpc - nfire, rpc)
        def _(r):
            pltpu.make_async_copy(
                a_sh.at[pl.ds(start_of(r), row_w)], out_hbm.at[i0 + r], csem
            ).wait()

    return emit_rows


def kernel(q, v, embeddings):
    q_len = int(q.shape[1])
    v_len = int(v.shape[1])
    max_len, d = int(embeddings.shape[0]), int(embeddings.shape[1])
    a_flat = _build_a_call(q_len, v_len, max_len, d)(embeddings.reshape(max_len * d))
    out = _emit_rows_call(q_len, v_len, max_len, d)(a_flat)
    return out.reshape(q_len, v_len, d)


# speedup vs baseline: 1.3786x; 1.3786x over previous
import functools

import jax
import jax.numpy as jnp
from jax import lax
from jax.experimental import pallas as pl
from jax.experimental.pallas import tpu as pltpu
from jax.experimental.pallas import tpu_sc as plsc

_NUM_CORES = 2
_NUM_SUBCORES = 16
_LANES = 16
_TC_LANES = 128


def _build_a_call(q_len, v_len, max_len, d):
    p = (max_len - 1) // 2
    off = (q_len - 1) - p
    a_rows = q_len + v_len
    nw = _NUM_CORES * _NUM_SUBCORES
    assert a_rows % nw == 0
    bpw = a_rows // nw

    mesh = plsc.VectorSubcoreMesh(core_axis_name="c", subcore_axis_name="s")

    @functools.partial(
        pl.kernel,
        out_type=jax.ShapeDtypeStruct((a_rows * d,), jnp.float32),
        mesh=mesh,
        compiler_params=pltpu.CompilerParams(use_tc_tiling_on_sc=False),
        scratch_types=[
            pltpu.VMEM((max_len * d,), jnp.float32),
            pltpu.VMEM((bpw * d,), jnp.float32),
            pltpu.SemaphoreType.DMA,
        ],
    )
    def build_a(emb_hbm, a_hbm, emb_v, build_v, sem):
        wid = lax.axis_index("c") * _NUM_SUBCORES + lax.axis_index("s")
        pltpu.async_copy(emb_hbm, emb_v, sem).wait()
        bias = wid * bpw - off

        @pl.loop(0, bpw, step=4)
        def _(t):
            for u in range(4):
                k = jnp.minimum(jnp.maximum(bias + (t + u), 0), max_len - 1)
                for h in range(d // _LANES):
                    build_v[pl.ds((t + u) * d + h * _LANES, _LANES)] = (
                        emb_v[pl.ds(k * d + h * _LANES, _LANES)]
                    )

        pltpu.async_copy(build_v, a_hbm.at[pl.ds(wid * bpw * d, bpw * d)], sem).wait()

    return build_a


def _emit_rows_tc(q_len, v_len, max_len, d):
    a_rows2 = (q_len + v_len) * d // _TC_LANES
    wrows = v_len * d // _TC_LANES
    shift = d % _TC_LANES
    nph = _TC_LANES // shift
    rpb = 8
    assert q_len % rpb == 0 and rpb % nph == 0

    def body(a_ref, o_ref, b_ref):
        @pl.when(pl.program_id(0) == 0)
        def _():
            av = a_ref[...]
            nxt = jnp.concatenate([av[1:], av[:1]], axis=0)
            b_ref[0] = av
            for ph in range(1, nph):
                b_ref[ph] = jnp.concatenate(
                    [av[:, ph * shift:], nxt[:, : ph * shift]], axis=1
                )

        g = pl.program_id(0)
        for r in range(rpb):
            i = g * rpb + r
            ph = ((q_len - 1) - r) % nph
            qrow = (((q_len - 1) - i) * d - shift * ph) // _TC_LANES
            o_ref[r] = b_ref[ph, pl.ds(qrow, wrows), :]

    return pl.pallas_call(
        body,
        grid=(q_len // rpb,),
        in_specs=[
            pl.BlockSpec((a_rows2, _TC_LANES), lambda g: (0, 0)),
        ],
        out_specs=pl.BlockSpec((rpb, wrows, _TC_LANES), lambda g: (g, 0, 0)),
        out_shape=jax.ShapeDtypeStruct((q_len, wrows, _TC_LANES), jnp.float32),
        scratch_shapes=[pltpu.VMEM((nph, a_rows2, _TC_LANES), jnp.float32)],
    )


def kernel(q, v, embeddings):
    q_len = int(q.shape[1])
    v_len = int(v.shape[1])
    max_len, d = int(embeddings.shape[0]), int(embeddings.shape[1])
    a_flat = _build_a_call(q_len, v_len, max_len, d)(embeddings.reshape(max_len * d))
    a2d = a_flat.reshape((q_len + v_len) * d // _TC_LANES, _TC_LANES)
    out = _emit_rows_tc(q_len, v_len, max_len, d)(a2d)
    return out.reshape(q_len, v_len, d)
